# baseline (speedup 1.0000x reference)
import jax
import jax.numpy as jnp
from jax.experimental import pallas as pl

N_LAYERS = 4
TOWERS = 8
D = 128
F_OUT = 16
N_GRAPHS = 64


def _pre_mlp_body(h_ref, w_ref, b_ref, o_ref):
    o_ref[...] = (
        jnp.dot(h_ref[...], w_ref[...], preferred_element_type=jnp.float32)
        + b_ref[...]
    )


def _pre_mlp(h_cat, preW, preb):
    E, K = h_cat.shape
    M = preW.shape[1]
    B = 2000
    return pl.pallas_call(
        _pre_mlp_body,
        grid=(E // B,),
        in_specs=[
            pl.BlockSpec((B, K), lambda i: (i, 0)),
            pl.BlockSpec((K, M), lambda i: (0, 0)),
            pl.BlockSpec((1, M), lambda i: (0, 0)),
        ],
        out_specs=pl.BlockSpec((B, M), lambda i: (i, 0)),
        out_shape=jax.ShapeDtypeStruct((E, M), jnp.float32),
    )(h_cat, preW, preb)


def _conv(x, src, dst, ea, preW, preb, postW, postb, linW, linb, deg, avg_log):
    n = x.shape[0]
    h = jnp.concatenate([x[dst], x[src], ea], axis=-1)
    w2 = jnp.transpose(preW, (1, 0, 2)).reshape(3 * D, TOWERS * D)
    msg = _pre_mlp(h, w2, preb.reshape(1, TOWERS * D))
    msg = msg.reshape(-1, TOWERS, D)
    denom = jnp.clip(deg, 1.0, None)[:, None, None]
    mean = jax.ops.segment_sum(msg, dst, num_segments=n) / denom
    mean2 = jax.ops.segment_sum(msg * msg, dst, num_segments=n) / denom
    std = jnp.sqrt(jax.nn.relu(mean2 - mean * mean) + 1e-5)
    has = (deg > 0)[:, None, None]
    mn = jnp.where(has, jax.ops.segment_min(msg, dst, num_segments=n), 0.0)
    mx = jnp.where(has, jax.ops.segment_max(msg, dst, num_segments=n), 0.0)
    agg = jnp.concatenate([mean, mn, mx, std], axis=-1)
    dcl = jnp.clip(deg, 1.0, None)[:, None, None]
    amp = agg * (jnp.log(dcl + 1.0) / avg_log)
    att = agg * (avg_log / jnp.log(dcl + 1.0))
    out = jnp.concatenate([agg, amp, att], axis=-1)
    xt = jnp.broadcast_to(x[:, None, :], (n, TOWERS, x.shape[-1]))
    out = jnp.concatenate([xt, out], axis=-1)
    out = jnp.einsum('nti,tio->nto', out, postW) + postb[None]
    out = out.reshape(n, TOWERS * F_OUT)
    return out @ linW + linb


def _bn(x, gamma, beta):
    mu = jnp.mean(x, axis=0)
    var = jnp.var(x, axis=0)
    return (x - mu) / jnp.sqrt(var + 1e-5) * gamma + beta


def kernel(x, edge_index, edge_attr, batch, node_emb, edge_emb, enc_W, enc_b,
           pre_W, pre_b, post_W, post_b, lin_W, lin_b, bn_gamma, bn_beta,
           W1, b1, W2, b2, W3, b3):
    src = edge_index[0]
    dst = edge_index[1]
    h = node_emb[x]
    ea0 = edge_emb[edge_attr]
    deg = jax.ops.segment_sum(jnp.ones(dst.shape[0], jnp.float32), dst,
                              num_segments=h.shape[0])
    avg_log = jnp.mean(jnp.log(deg + 1.0))
    for l in range(N_LAYERS):
        ea = ea0 @ enc_W[l] + enc_b[l]
        h = _conv(h, src, dst, ea, pre_W[l], pre_b[l], post_W[l], post_b[l],
                  lin_W[l], lin_b[l], deg, avg_log)
        h = jax.nn.relu(_bn(h, bn_gamma[l], bn_beta[l]))
    pooled = jax.ops.segment_sum(h, batch, num_segments=N_GRAPHS)
    z = jax.nn.relu(pooled @ W1 + b1)
    z = jax.nn.relu(z @ W2 + b2)
    out = jax.nn.sigmoid(z @ W3 + b3)
    return (out, pooled)


# speedup vs baseline: 13.6513x; 13.6513x over previous
import functools

import jax
import jax.numpy as jnp
from jax import lax
from jax.experimental import pallas as pl
from jax.experimental.pallas import tpu as pltpu
from jax.experimental.pallas import tpu_sc as plsc

N_LAYERS = 4
TOWERS = 8
D = 128
F_OUT = 16
N_GRAPHS = 64
N_NODES = 10000
TD = TOWERS * D

_V = 160
_NCHUNK = 64
_NPAD = _V * _NCHUNK
_CH = 128
_FLT_BIG = 3.0e38


def _matmul_body(h_ref, w_ref, b_ref, o_ref):
    o_ref[...] = (
        jnp.dot(h_ref[...], w_ref[...], preferred_element_type=jnp.float32)
        + b_ref[...]
    )


def _edge_matmul(h_cat, W, b):
    E, K = h_cat.shape
    M = W.shape[1]
    B = 2000
    return pl.pallas_call(
        _matmul_body,
        grid=(E // B,),
        in_specs=[
            pl.BlockSpec((B, K), lambda i: (i, 0)),
            pl.BlockSpec((K, M), lambda i: (0, 0)),
            pl.BlockSpec((1, M), lambda i: (0, 0)),
        ],
        out_specs=pl.BlockSpec((B, M), lambda i: (i, 0)),
        out_shape=jax.ShapeDtypeStruct((E, M), jnp.float32),
    )(h_cat, W, b)


def _weff_body(encW_ref, preWf_ref, encb_ref, prebf_ref, wo_ref, bo_ref):
    we = preWf_ref[pl.ds(2 * D, D), :]
    wo_ref[pl.ds(0, 2 * D), :] = preWf_ref[pl.ds(0, 2 * D), :]
    wo_ref[pl.ds(2 * D, D), :] = jnp.dot(
        encW_ref[...], we, preferred_element_type=jnp.float32)
    bo_ref[...] = prebf_ref[...] + jnp.dot(
        encb_ref[...], we, preferred_element_type=jnp.float32)


def _weff(encW, preW_flat, enc_b, preb_flat):
    return pl.pallas_call(
        _weff_body,
        out_shape=(
            jax.ShapeDtypeStruct((3 * D, TD), jnp.float32),
            jax.ShapeDtypeStruct((1, TD), jnp.float32),
        ),
    )(encW, preW_flat, enc_b.reshape(1, D), preb_flat.reshape(1, TD))


def _sc_reduce(msg, dst_sorted, eb):
    info = plsc.get_sparse_core_info()
    NC = info.num_cores
    mesh = plsc.VectorSubcoreMesh(core_axis_name="c", subcore_axis_name="s")
    out_t = [jax.ShapeDtypeStruct((_NPAD, TD), jnp.float32)] * 4

    @functools.partial(
        pl.kernel, mesh=mesh, out_type=out_t,
        scratch_types=[
            pltpu.VMEM((_V + 1, D), jnp.float32),
            pltpu.VMEM((_V + 1, D), jnp.float32),
            pltpu.VMEM((_V + 1, D), jnp.float32),
            pltpu.VMEM((_V + 1, D), jnp.float32),
            pltpu.VMEM((_CH, D), jnp.float32),
            pltpu.VMEM((_CH + 16,), jnp.int32),
            pltpu.VMEM((72,), jnp.int32),
        ])
    def k(msg_h, dst_h, eb_h, sum_h, sq_h, mn_h, mx_h,
          accs, accq, accn, accx, msgv, dv, ebv):
        wid = lax.axis_index("s") * NC + lax.axis_index("c")
        pltpu.sync_copy(eb_h, ebv)
        zero = jnp.zeros((16,), jnp.float32)
        big = jnp.full((16,), _FLT_BIG, jnp.float32)
        one = jnp.minimum(ebv[pl.ds(0, 16)][0] * 0 + 1, 1)

        def task_body(tk, _):
            kk = tk // 8
            t = tk % 8
            c = wid * 2 + kk
            nb = c * _V
            ebvec = ebv[pl.ds(c, 16)]
            lo = ebvec[0]
            hi = ebvec[1]
            j0 = lo // _CH
            j1 = (hi + _CH - 1) // _CH

            def init_body(r, _):
                for s in range(8):
                    sl = pl.ds(s * 16, 16)
                    accs[r, sl] = zero
                    accq[r, sl] = zero
                    accn[r, sl] = big
                    accx[r, sl] = -big
                return 0

            lax.fori_loop(0, one * (_V + 1), init_body, 0)

            def chunk_body(j, _):
                e0 = j * _CH
                pltpu.sync_copy(dst_h.at[pl.ds(e0, _CH)], dv.at[pl.ds(0, _CH)])
                pltpu.sync_copy(
                    msg_h.at[pl.ds(e0, _CH), pl.ds(t * D, D)], msgv)

                def edge_body(i, _):
                    li = dv[pl.ds(i, 16)][0] - nb
                    ok = (li >= 0) & (li < _V)
                    r = jnp.where(ok, li, _V)
                    for s in range(8):
                        sl = pl.ds(s * 16, 16)
                        m = msgv[i, sl]
                        accs[r, sl] = accs[r, sl] + m
                        accq[r, sl] = accq[r, sl] + m * m
                        accn[r, sl] = jnp.minimum(accn[r, sl], m)
                        accx[r, sl] = jnp.maximum(accx[r, sl], m)
                    return 0

                lax.fori_loop(0, one * _CH, edge_body, 0)
                return 0

            lax.fori_loop(j0, j1, chunk_body, 0)

            for oref, aref in ((sum_h, accs), (sq_h, accq),
                               (mn_h, accn), (mx_h, accx)):
                pltpu.sync_copy(
                    aref.at[pl.ds(0, _V), :],
                    oref.at[pl.ds(nb, _V), pl.ds(t * D, D)])
            return 0

        lax.fori_loop(0, one * 16, task_body, 0)

    return k(msg, dst_sorted, eb)


def kernel(x, edge_index, edge_attr, batch, node_emb, edge_emb, enc_W, enc_b,
           pre_W, pre_b, post_W, post_b, lin_W, lin_b, bn_gamma, bn_beta,
           W1, b1, W2, b2, W3, b3):
    src = edge_index[0]
    dst = edge_index[1]
    n = N_NODES

    perm = jnp.argsort(dst)
    ds_ = dst[perm].astype(jnp.int32)
    ss_ = src[perm].astype(jnp.int32)
    eas_ = edge_attr[perm]
    row_ptr = jnp.searchsorted(ds_, jnp.arange(n + 1)).astype(jnp.int32)
    deg = (row_ptr[1:] - row_ptr[:-1]).astype(jnp.float32)
    eb = row_ptr[jnp.clip(jnp.arange(65) * _V, 0, n)].astype(jnp.int32)
    eb = jnp.concatenate([eb, jnp.zeros((7,), jnp.int32)])

    h = node_emb[x]
    ea0s = edge_emb[eas_]
    avg_log = jnp.mean(jnp.log(deg + 1.0))

    denom = jnp.clip(deg, 1.0, None)[:, None, None]
    has = (deg > 0)[:, None, None]
    dcl = jnp.clip(deg, 1.0, None)[:, None, None]
    ampf = jnp.log(dcl + 1.0) / avg_log
    attf = avg_log / jnp.log(dcl + 1.0)

    for l in range(N_LAYERS):
        preW_flat = jnp.transpose(pre_W[l], (1, 0, 2)).reshape(3 * D, TD)
        preb_flat = pre_b[l].reshape(1, TD)
        W_eff, b_eff = _weff(enc_W[l], preW_flat, enc_b[l], preb_flat)
        h_cat = jnp.concatenate([h[ds_], h[ss_], ea0s], axis=-1)
        msg = _edge_matmul(h_cat, W_eff, b_eff)
        sums, sqs, mns, mxs = _sc_reduce(msg, ds_, eb)
        sums = sums[:n].reshape(n, TOWERS, D)
        sqs = sqs[:n].reshape(n, TOWERS, D)
        mns = mns[:n].reshape(n, TOWERS, D)
        mxs = mxs[:n].reshape(n, TOWERS, D)
        mean = sums / denom
        mean2 = sqs / denom
        std = jnp.sqrt(jax.nn.relu(mean2 - mean * mean) + 1e-5)
        mn = jnp.where(has, mns, 0.0)
        mx = jnp.where(has, mxs, 0.0)
        agg = jnp.concatenate([mean, mn, mx, std], axis=-1)
        out = jnp.concatenate([agg, agg * ampf, agg * attf], axis=-1)
        xt = jnp.broadcast_to(h[:, None, :], (n, TOWERS, D))
        out = jnp.concatenate([xt, out], axis=-1)
        out = jnp.einsum('nti,tio->nto', out, post_W[l]) + post_b[l][None]
        out = out.reshape(n, TOWERS * F_OUT)
        hpre = out @ lin_W[l] + lin_b[l]
        mu = jnp.mean(hpre, axis=0)
        var = jnp.var(hpre, axis=0)
        h = jax.nn.relu(
            (hpre - mu) / jnp.sqrt(var + 1e-5) * bn_gamma[l] + bn_beta[l])

    pooled = jax.ops.segment_sum(h, batch, num_segments=N_GRAPHS)
    z = jax.nn.relu(pooled @ W1 + b1)
    z = jax.nn.relu(z @ W2 + b2)
    out = jax.nn.sigmoid(z @ W3 + b3)
    return (out, pooled)
